# baseline (speedup 1.0000x reference)
import functools

import jax
import jax.numpy as jnp
from jax import lax
from jax.experimental import pallas as pl
from jax.experimental.pallas import tpu as pltpu
from jax.experimental.pallas import tpu_sc as plsc

N_TOK = 2048
D = 768
H = 3072
E = 16

TM = 256
NMT = N_TOK // TM
T_MAX = NMT + E - 1

L = 16
SORT_NW = 16
SORT_CHUNK = N_TOK // SORT_NW
COMB_NW = 32
COMB_CHUNK = N_TOK // COMB_NW


def _lane(v, j):
    lanes = lax.iota(jnp.int32, L)
    return jnp.sum(jnp.where(lanes == j, v, 0), axis=0)


def _dispatch_body(eids_hbm, x_hbm, xs_out, inv_out, meta_out,
                   eid_v, hist_v, hrow_v, pos_v, tid_v, perm_v, rows_v,
                   ft_v, ts_v, st_v, en_v, meta_v, sh_hist, sh_perm, sem):
    wid = lax.axis_index("s")
    base = wid * SORT_CHUNK
    lanes = lax.iota(jnp.int32, L)

    if True:
        pltpu.sync_copy(eids_hbm.at[pl.ds(base, SORT_CHUNK)], eid_v)
        hv = jnp.zeros((L,), jnp.int32)
        keys = []
        for k in range(SORT_CHUNK // L):
            kv = eid_v[pl.ds(k * L, L)]
            for j in range(L):
                e = jnp.sum(jnp.where(lanes == j, kv, 0), axis=0)
                onehot = (lanes == e)
                hv = hv + onehot.astype(jnp.int32)
                keys.append(onehot)
        hrow_v[...] = hv
        pltpu.sync_copy(hrow_v, sh_hist.at[wid])
        plsc.subcore_barrier()

        pltpu.sync_copy(sh_hist, hist_v)
        tot = jnp.zeros((L,), jnp.int32)
        pre = jnp.zeros((L,), jnp.int32)
        for w in range(SORT_NW):
            rw = hist_v[w, :]
            tot = tot + rw
            pre = pre + jnp.where(jnp.int32(w) < wid, rw, 0)
        excl = jnp.cumsum(tot, axis=0) - tot
        start = excl + pre

        for k in range(SORT_CHUNK // L):
            posv = jnp.zeros((L,), jnp.int32)
            for j in range(L):
                onehot = keys[k * L + j]
                p = jnp.sum(jnp.where(onehot, start, 0), axis=0)
                start = start + onehot.astype(jnp.int32)
                posv = jnp.where(lanes == j, p, posv)
            pos_v[pl.ds(k * L, L)] = posv
            tid_v[pl.ds(k * L, L)] = base + k * L + lanes
        pltpu.sync_copy(tid_v, sh_perm.at[pos_v])
        plsc.subcore_barrier()

        pltpu.sync_copy(sh_perm.at[pl.ds(base, SORT_CHUNK)], perm_v)
        pltpu.async_copy(x_hbm.at[perm_v], rows_v, sem).wait()
        pltpu.sync_copy(rows_v, xs_out.at[pl.ds(base, SORT_CHUNK)])
        pltpu.sync_copy(pos_v, inv_out.at[pl.ds(base, SORT_CHUNK)])

        @pl.when(wid == 0)
        def _():
            ends = jnp.cumsum(tot, axis=0)
            starts = ends - tot
            ft = starts // TM
            lt = jnp.where(tot > 0, (ends - 1) // TM, ft)
            ntl = jnp.where(tot > 0, lt - ft + 1, 0)
            te = jnp.cumsum(ntl, axis=0)
            ts = te - ntl
            total = _lane(te, E - 1)
            ft_v[...] = ft
            ts_v[...] = ts
            st_v[...] = starts
            en_v[...] = ends
            last = total - 1
            g_last = jnp.sum(jnp.where(te <= last, 1, 0), axis=0)
            m_last = _lane(ft, g_last) + (last - _lane(ts, g_last))
            for half in range(2):
                sv = half * L + lanes
                gv = jnp.zeros((L,), jnp.int32)
                for gp in range(E):
                    gv = gv + jnp.where(_lane(te, gp) <= sv, 1, 0)
                gv = jnp.minimum(gv, E - 1)
                mv = plsc.load_gather(ft_v, [gv]) + sv - plsc.load_gather(ts_v, [gv])
                lov = jnp.maximum(plsc.load_gather(st_v, [gv]), mv * TM)
                hiv = jnp.minimum(plsc.load_gather(en_v, [gv]), (mv + 1) * TM)
                valid = sv < total
                mv = jnp.where(valid, mv, m_last)
                gv = jnp.where(valid, gv, g_last)
                lov = jnp.where(valid, lov, 0)
                hiv = jnp.where(valid, hiv, 0)
                meta_v[0, pl.ds(half * L, L)] = mv
                meta_v[1, pl.ds(half * L, L)] = gv
                meta_v[2, pl.ds(half * L, L)] = lov
                meta_v[3, pl.ds(half * L, L)] = hiv
            pltpu.sync_copy(meta_v, meta_out)


def _dispatch(e_ids, x_flat):
    mesh = plsc.VectorSubcoreMesh(
        core_axis_name="c", subcore_axis_name="s", num_cores=1)
    fn = functools.partial(
        pl.kernel,
        out_type=[
            jax.ShapeDtypeStruct((N_TOK, D), jnp.float32),
            jax.ShapeDtypeStruct((N_TOK,), jnp.int32),
            jax.ShapeDtypeStruct((4, 2 * L), jnp.int32),
        ],
        scratch_types=[
            pltpu.VMEM((SORT_CHUNK,), jnp.int32),
            pltpu.VMEM((SORT_NW, L), jnp.int32),
            pltpu.VMEM((L,), jnp.int32),
            pltpu.VMEM((SORT_CHUNK,), jnp.int32),
            pltpu.VMEM((SORT_CHUNK,), jnp.int32),
            pltpu.VMEM((SORT_CHUNK,), jnp.int32),
            pltpu.VMEM((SORT_CHUNK, D), jnp.float32),
            pltpu.VMEM((L,), jnp.int32),
            pltpu.VMEM((L,), jnp.int32),
            pltpu.VMEM((L,), jnp.int32),
            pltpu.VMEM((L,), jnp.int32),
            pltpu.VMEM((4, 2 * L), jnp.int32),
            pltpu.VMEM_SHARED((SORT_NW, L), jnp.int32),
            pltpu.VMEM_SHARED((N_TOK,), jnp.int32),
            pltpu.SemaphoreType.DMA,
        ],
        mesh=mesh,
        compiler_params=pltpu.CompilerParams(needs_layout_passes=False),
    )(_dispatch_body)
    return fn(e_ids, x_flat)


def _combine_body(y_hbm, inv_hbm, out_hbm, iv_v, rows_v, sem):
    wid = lax.axis_index("s") * 2 + lax.axis_index("c")
    base = wid * COMB_CHUNK
    pltpu.sync_copy(inv_hbm.at[pl.ds(base, COMB_CHUNK)], iv_v)
    pltpu.async_copy(y_hbm.at[iv_v], rows_v, sem).wait()
    pltpu.sync_copy(rows_v, out_hbm.at[pl.ds(base, COMB_CHUNK)])


def _combine(y_sorted, inv):
    mesh = plsc.VectorSubcoreMesh(core_axis_name="c", subcore_axis_name="s")
    fn = functools.partial(
        pl.kernel,
        out_type=jax.ShapeDtypeStruct((N_TOK, D), jnp.float32),
        scratch_types=[
            pltpu.VMEM((COMB_CHUNK,), jnp.int32),
            pltpu.VMEM((COMB_CHUNK, D), jnp.float32),
            pltpu.SemaphoreType.DMA,
        ],
        mesh=mesh,
        compiler_params=pltpu.CompilerParams(needs_layout_passes=False),
    )(_combine_body)
    return fn(y_sorted, inv)


def _router_body(x_ref, wr_ref, out_ref):
    logits = lax.dot_general(
        x_ref[...], wr_ref[...], (((1,), (1,)), ((), ())),
        preferred_element_type=jnp.float32)
    am = jnp.argmax(logits, axis=1).astype(jnp.int32)
    out_ref[...] = am.reshape(TM, 1)


def _router_argmax(x_flat, Wr):
    return pl.pallas_call(
        _router_body,
        grid=(NMT,),
        in_specs=[
            pl.BlockSpec((TM, D), lambda i: (i, 0)),
            pl.BlockSpec((E, D), lambda i: (0, 0)),
        ],
        out_specs=pl.BlockSpec((TM, 1), lambda i: (i, 0)),
        out_shape=jax.ShapeDtypeStruct((N_TOK, 1), jnp.int32),
    )(x_flat, Wr)


def _ffn_body(meta_ref, x_ref, w1_ref, b1_ref, w2_ref, b2_ref, out_ref):
    t = pl.program_id(0)
    m = meta_ref[0, t]
    lo = meta_ref[2, t]
    hi = meta_ref[3, t]
    prev_m = meta_ref[0, jnp.maximum(t - 1, 0)]
    first = (t == 0) | (m != prev_m)

    @pl.when(first)
    def _():
        out_ref[...] = jnp.zeros_like(out_ref)

    @pl.when(lo < hi)
    def _():
        xb = x_ref[...]
        w1 = w1_ref[0]
        hb = lax.dot_general(xb, w1, (((1,), (1,)), ((), ())),
                             preferred_element_type=jnp.float32)
        hb = jnp.maximum(hb + b1_ref[0, 0][None, :], 0.0)
        rows = m * TM + lax.broadcasted_iota(jnp.int32, (TM, 1), 0)
        mask = (rows >= lo) & (rows < hi)
        hb = jnp.where(mask, hb, 0.0)
        w2 = w2_ref[0]
        yb = lax.dot_general(hb, w2, (((1,), (1,)), ((), ())),
                             preferred_element_type=jnp.float32)
        yb = yb + jnp.where(mask, b2_ref[0, 0][None, :], 0.0)
        out_ref[...] += yb


def _grouped_ffn(x_sorted, W1, b1, W2, b2, meta):
    grid_spec = pltpu.PrefetchScalarGridSpec(
        num_scalar_prefetch=1,
        grid=(T_MAX,),
        in_specs=[
            pl.BlockSpec((TM, D), lambda t, md: (md[0, t], 0)),
            pl.BlockSpec((1, H, D), lambda t, md: (md[1, t], 0, 0)),
            pl.BlockSpec((1, 1, H), lambda t, md: (md[1, t], 0, 0)),
            pl.BlockSpec((1, D, H), lambda t, md: (md[1, t], 0, 0)),
            pl.BlockSpec((1, 1, D), lambda t, md: (md[1, t], 0, 0)),
        ],
        out_specs=pl.BlockSpec((TM, D), lambda t, md: (md[0, t], 0)),
    )
    return pl.pallas_call(
        _ffn_body,
        grid_spec=grid_spec,
        out_shape=jax.ShapeDtypeStruct((N_TOK, D), jnp.float32),
    )(meta, x_sorted, W1, b1.reshape(E, 1, H), W2, b2.reshape(E, 1, D))


def kernel(x, Wr, W1, b1, W2, b2):
    Bc, Tc, C = x.shape
    x_flat = x.reshape(-1, C)
    e_ids = _router_argmax(x_flat, Wr).reshape(N_TOK)
    x_sorted, inv, meta = _dispatch(e_ids, x_flat)
    y_sorted = _grouped_ffn(x_sorted, W1, b1, W2, b2, meta)
    out = _combine(y_sorted, inv)
    return out.reshape(Bc, Tc, C)


# speedup vs baseline: 1.0115x; 1.0115x over previous
import functools

import jax
import jax.numpy as jnp
from jax import lax
from jax.experimental import pallas as pl
from jax.experimental.pallas import tpu as pltpu
from jax.experimental.pallas import tpu_sc as plsc

N_TOK = 2048
D = 768
H = 3072
E = 16

TM = 256
NMT = N_TOK // TM
T_MAX = NMT + E - 1

L = 16
SORT_NW = 16
SORT_CHUNK = N_TOK // SORT_NW
COMB_NW = 32
COMB_CHUNK = N_TOK // COMB_NW


def _lane(v, j):
    lanes = lax.iota(jnp.int32, L)
    return jnp.sum(jnp.where(lanes == j, v, 0), axis=0)


def _dispatch_body(eids_hbm, x_hbm, xs_out, inv_out, meta_out,
                   eid_v, hist_v, hrow_v, pos_v, tid_v, perm_v, rows_v,
                   ft_v, ts_v, st_v, en_v, meta_v, sh_hist, sh_perm,
                   sem, wsem):
    wid = lax.axis_index("s")
    base = wid * SORT_CHUNK
    lanes = lax.iota(jnp.int32, L)

    if True:
        pltpu.sync_copy(eids_hbm.at[pl.ds(base, SORT_CHUNK)], eid_v)
        hv = jnp.zeros((L,), jnp.int32)
        keys = []
        for k in range(SORT_CHUNK // L):
            kv = eid_v[pl.ds(k * L, L)]
            for j in range(L):
                e = jnp.sum(jnp.where(lanes == j, kv, 0), axis=0)
                onehot = (lanes == e)
                hv = hv + onehot.astype(jnp.int32)
                keys.append(onehot)
        hrow_v[...] = hv
        pltpu.sync_copy(hrow_v, sh_hist.at[wid])
        plsc.subcore_barrier()

        pltpu.sync_copy(sh_hist, hist_v)
        tot = jnp.zeros((L,), jnp.int32)
        pre = jnp.zeros((L,), jnp.int32)
        for w in range(SORT_NW):
            rw = hist_v[w, :]
            tot = tot + rw
            pre = pre + jnp.where(jnp.int32(w) < wid, rw, 0)
        excl = jnp.cumsum(tot, axis=0) - tot
        start = excl + pre

        for k in range(SORT_CHUNK // L):
            posv = jnp.zeros((L,), jnp.int32)
            for j in range(L):
                onehot = keys[k * L + j]
                p = jnp.sum(jnp.where(onehot, start, 0), axis=0)
                start = start + onehot.astype(jnp.int32)
                posv = jnp.where(lanes == j, p, posv)
            pos_v[pl.ds(k * L, L)] = posv
            tid_v[pl.ds(k * L, L)] = base + k * L + lanes
        pltpu.sync_copy(tid_v, sh_perm.at[pos_v])
        plsc.subcore_barrier()

        pltpu.sync_copy(sh_perm.at[pl.ds(base, SORT_CHUNK)], perm_v)
        nch = SORT_CHUNK // L
        gcopies = []
        for k in range(nch):
            ivk = perm_v[pl.ds(k * L, L)]
            gcopies.append(pltpu.async_copy(
                x_hbm.at[ivk], rows_v.at[pl.ds(k * L, L)], sem))
        wcopies = []
        for k in range(nch):
            gcopies[k].wait()
            wcopies.append(pltpu.async_copy(
                rows_v.at[pl.ds(k * L, L)],
                xs_out.at[pl.ds(base + k * L, L)], wsem))
        pltpu.sync_copy(pos_v, inv_out.at[pl.ds(base, SORT_CHUNK)])
        for c in wcopies:
            c.wait()

        @pl.when(wid == 0)
        def _():
            ends = jnp.cumsum(tot, axis=0)
            starts = ends - tot
            ft = starts // TM
            lt = jnp.where(tot > 0, (ends - 1) // TM, ft)
            ntl = jnp.where(tot > 0, lt - ft + 1, 0)
            te = jnp.cumsum(ntl, axis=0)
            ts = te - ntl
            total = _lane(te, E - 1)
            ft_v[...] = ft
            ts_v[...] = ts
            st_v[...] = starts
            en_v[...] = ends
            last = total - 1
            g_last = jnp.sum(jnp.where(te <= last, 1, 0), axis=0)
            m_last = _lane(ft, g_last) + (last - _lane(ts, g_last))
            for half in range(2):
                sv = half * L + lanes
                gv = jnp.zeros((L,), jnp.int32)
                for gp in range(E):
                    gv = gv + jnp.where(_lane(te, gp) <= sv, 1, 0)
                gv = jnp.minimum(gv, E - 1)
                mv = plsc.load_gather(ft_v, [gv]) + sv - plsc.load_gather(ts_v, [gv])
                lov = jnp.maximum(plsc.load_gather(st_v, [gv]), mv * TM)
                hiv = jnp.minimum(plsc.load_gather(en_v, [gv]), (mv + 1) * TM)
                valid = sv < total
                mv = jnp.where(valid, mv, m_last)
                gv = jnp.where(valid, gv, g_last)
                lov = jnp.where(valid, lov, 0)
                hiv = jnp.where(valid, hiv, 0)
                meta_v[0, pl.ds(half * L, L)] = mv
                meta_v[1, pl.ds(half * L, L)] = gv
                meta_v[2, pl.ds(half * L, L)] = lov
                meta_v[3, pl.ds(half * L, L)] = hiv
            pltpu.sync_copy(meta_v, meta_out)


def _dispatch(e_ids, x_flat):
    mesh = plsc.VectorSubcoreMesh(
        core_axis_name="c", subcore_axis_name="s", num_cores=1)
    fn = functools.partial(
        pl.kernel,
        out_type=[
            jax.ShapeDtypeStruct((N_TOK, D), jnp.float32),
            jax.ShapeDtypeStruct((N_TOK,), jnp.int32),
            jax.ShapeDtypeStruct((4, 2 * L), jnp.int32),
        ],
        scratch_types=[
            pltpu.VMEM((SORT_CHUNK,), jnp.int32),
            pltpu.VMEM((SORT_NW, L), jnp.int32),
            pltpu.VMEM((L,), jnp.int32),
            pltpu.VMEM((SORT_CHUNK,), jnp.int32),
            pltpu.VMEM((SORT_CHUNK,), jnp.int32),
            pltpu.VMEM((SORT_CHUNK,), jnp.int32),
            pltpu.VMEM((SORT_CHUNK, D), jnp.float32),
            pltpu.VMEM((L,), jnp.int32),
            pltpu.VMEM((L,), jnp.int32),
            pltpu.VMEM((L,), jnp.int32),
            pltpu.VMEM((L,), jnp.int32),
            pltpu.VMEM((4, 2 * L), jnp.int32),
            pltpu.VMEM_SHARED((SORT_NW, L), jnp.int32),
            pltpu.VMEM_SHARED((N_TOK,), jnp.int32),
            pltpu.SemaphoreType.DMA,
            pltpu.SemaphoreType.DMA,
        ],
        mesh=mesh,
        compiler_params=pltpu.CompilerParams(needs_layout_passes=False),
    )(_dispatch_body)
    return fn(e_ids, x_flat)


def _combine_body(y_hbm, inv_hbm, out_hbm, iv_v, rows_v, sem, wsem):
    wid = lax.axis_index("s") * 2 + lax.axis_index("c")
    base = wid * COMB_CHUNK
    pltpu.sync_copy(inv_hbm.at[pl.ds(base, COMB_CHUNK)], iv_v)
    nch = COMB_CHUNK // L
    gcopies = []
    for k in range(nch):
        ivk = iv_v[pl.ds(k * L, L)]
        gcopies.append(pltpu.async_copy(
            y_hbm.at[ivk], rows_v.at[pl.ds(k * L, L)], sem))
    wcopies = []
    for k in range(nch):
        gcopies[k].wait()
        wcopies.append(pltpu.async_copy(
            rows_v.at[pl.ds(k * L, L)],
            out_hbm.at[pl.ds(base + k * L, L)], wsem))
    for c in wcopies:
        c.wait()


def _combine(y_sorted, inv):
    mesh = plsc.VectorSubcoreMesh(core_axis_name="c", subcore_axis_name="s")
    fn = functools.partial(
        pl.kernel,
        out_type=jax.ShapeDtypeStruct((N_TOK, D), jnp.float32),
        scratch_types=[
            pltpu.VMEM((COMB_CHUNK,), jnp.int32),
            pltpu.VMEM((COMB_CHUNK, D), jnp.float32),
            pltpu.SemaphoreType.DMA,
            pltpu.SemaphoreType.DMA,
        ],
        mesh=mesh,
        compiler_params=pltpu.CompilerParams(needs_layout_passes=False),
    )(_combine_body)
    return fn(y_sorted, inv)


def _router_body(x_ref, wr_ref, out_ref):
    logits = lax.dot_general(
        x_ref[...], wr_ref[...], (((1,), (1,)), ((), ())),
        preferred_element_type=jnp.float32)
    am = jnp.argmax(logits, axis=1).astype(jnp.int32)
    out_ref[...] = am.reshape(TM, 1)


def _router_argmax(x_flat, Wr):
    return pl.pallas_call(
        _router_body,
        grid=(NMT,),
        in_specs=[
            pl.BlockSpec((TM, D), lambda i: (i, 0)),
            pl.BlockSpec((E, D), lambda i: (0, 0)),
        ],
        out_specs=pl.BlockSpec((TM, 1), lambda i: (i, 0)),
        out_shape=jax.ShapeDtypeStruct((N_TOK, 1), jnp.int32),
    )(x_flat, Wr)


def _ffn_body(meta_ref, x_ref, w1_ref, b1_ref, w2_ref, b2_ref, out_ref):
    t = pl.program_id(0)
    m = meta_ref[0, t]
    lo = meta_ref[2, t]
    hi = meta_ref[3, t]
    prev_m = meta_ref[0, jnp.maximum(t - 1, 0)]
    first = (t == 0) | (m != prev_m)

    @pl.when(first)
    def _():
        out_ref[...] = jnp.zeros_like(out_ref)

    @pl.when(lo < hi)
    def _():
        xb = x_ref[...]
        w1 = w1_ref[0]
        hb = lax.dot_general(xb, w1, (((1,), (1,)), ((), ())),
                             preferred_element_type=jnp.float32)
        hb = jnp.maximum(hb + b1_ref[0, 0][None, :], 0.0)
        rows = m * TM + lax.broadcasted_iota(jnp.int32, (TM, 1), 0)
        mask = (rows >= lo) & (rows < hi)
        hb = jnp.where(mask, hb, 0.0)
        w2 = w2_ref[0]
        yb = lax.dot_general(hb, w2, (((1,), (1,)), ((), ())),
                             preferred_element_type=jnp.float32)
        yb = yb + jnp.where(mask, b2_ref[0, 0][None, :], 0.0)
        out_ref[...] += yb


def _grouped_ffn(x_sorted, W1, b1, W2, b2, meta):
    grid_spec = pltpu.PrefetchScalarGridSpec(
        num_scalar_prefetch=1,
        grid=(T_MAX,),
        in_specs=[
            pl.BlockSpec((TM, D), lambda t, md: (md[0, t], 0)),
            pl.BlockSpec((1, H, D), lambda t, md: (md[1, t], 0, 0)),
            pl.BlockSpec((1, 1, H), lambda t, md: (md[1, t], 0, 0)),
            pl.BlockSpec((1, D, H), lambda t, md: (md[1, t], 0, 0)),
            pl.BlockSpec((1, 1, D), lambda t, md: (md[1, t], 0, 0)),
        ],
        out_specs=pl.BlockSpec((TM, D), lambda t, md: (md[0, t], 0)),
    )
    return pl.pallas_call(
        _ffn_body,
        grid_spec=grid_spec,
        out_shape=jax.ShapeDtypeStruct((N_TOK, D), jnp.float32),
    )(meta, x_sorted, W1, b1.reshape(E, 1, H), W2, b2.reshape(E, 1, D))


def kernel(x, Wr, W1, b1, W2, b2):
    Bc, Tc, C = x.shape
    x_flat = x.reshape(-1, C)
    e_ids = _router_argmax(x_flat, Wr).reshape(N_TOK)
    x_sorted, inv, meta = _dispatch(e_ids, x_flat)
    y_sorted = _grouped_ffn(x_sorted, W1, b1, W2, b2, meta)
    out = _combine(y_sorted, inv)
    return out.reshape(Bc, Tc, C)


# speedup vs baseline: 1.0117x; 1.0002x over previous
import functools

import jax
import jax.numpy as jnp
from jax import lax
from jax.experimental import pallas as pl
from jax.experimental.pallas import tpu as pltpu
from jax.experimental.pallas import tpu_sc as plsc

N_TOK = 2048
D = 768
H = 3072
E = 16

TM = 256
NMT = N_TOK // TM
T_MAX = NMT + E - 1

L = 16
SORT_NW = 16
SORT_CHUNK = N_TOK // SORT_NW
COMB_NW = 32
COMB_CHUNK = N_TOK // COMB_NW


def _lane(v, j):
    lanes = lax.iota(jnp.int32, L)
    return jnp.sum(jnp.where(lanes == j, v, 0), axis=0)


def _dispatch_body(eids_hbm, x_hbm, xs_out, inv_out, meta_out,
                   eid_v, hist_v, hrow_v, pos_v, tid_v, perm_v, rows_v,
                   ft_v, ts_v, st_v, en_v, meta_v, sh_hist, sh_perm,
                   sem, wsem):
    wid = lax.axis_index("s")
    base = wid * SORT_CHUNK
    lanes = lax.iota(jnp.int32, L)

    pltpu.sync_copy(eids_hbm.at[pl.ds(base, SORT_CHUNK)], eid_v)
    hv = jnp.zeros((L,), jnp.int32)
    keys = []
    for k in range(SORT_CHUNK // L):
        kv = eid_v[pl.ds(k * L, L)]
        for j in range(L):
            e = jnp.sum(jnp.where(lanes == j, kv, 0), axis=0)
            onehot = (lanes == e)
            hv = hv + onehot.astype(jnp.int32)
            keys.append(onehot)
    hrow_v[...] = hv
    pltpu.sync_copy(hrow_v, sh_hist.at[wid])
    plsc.subcore_barrier()

    pltpu.sync_copy(sh_hist, hist_v)
    tot = jnp.zeros((L,), jnp.int32)
    pre = jnp.zeros((L,), jnp.int32)
    for w in range(SORT_NW):
        rw = hist_v[w, :]
        tot = tot + rw
        pre = pre + jnp.where(jnp.int32(w) < wid, rw, 0)
    excl = jnp.cumsum(tot, axis=0) - tot
    start = excl + pre

    for k in range(SORT_CHUNK // L):
        posv = jnp.zeros((L,), jnp.int32)
        for j in range(L):
            onehot = keys[k * L + j]
            p = jnp.sum(jnp.where(onehot, start, 0), axis=0)
            start = start + onehot.astype(jnp.int32)
            posv = jnp.where(lanes == j, p, posv)
        pos_v[pl.ds(k * L, L)] = posv
        tid_v[pl.ds(k * L, L)] = base + k * L + lanes
    pltpu.sync_copy(tid_v, sh_perm.at[pos_v])
    plsc.subcore_barrier()

    pltpu.sync_copy(sh_perm.at[pl.ds(base, SORT_CHUNK)], perm_v)
    nch = SORT_CHUNK // L
    gcopies = []
    for k in range(nch):
        ivk = perm_v[pl.ds(k * L, L)]
        gcopies.append(pltpu.async_copy(
            x_hbm.at[ivk], rows_v.at[pl.ds(k * L, L)], sem))
    wcopies = []
    for k in range(nch):
        gcopies[k].wait()
        wcopies.append(pltpu.async_copy(
            rows_v.at[pl.ds(k * L, L)],
            xs_out.at[pl.ds(base + k * L, L)], wsem))
    pltpu.sync_copy(pos_v, inv_out.at[pl.ds(base, SORT_CHUNK)])
    for c in wcopies:
        c.wait()

    @pl.when(wid == 0)
    def _():
        ends = jnp.cumsum(tot, axis=0)
        starts = ends - tot
        ft = starts // TM
        lt = jnp.where(tot > 0, (ends - 1) // TM, ft)
        ntl = jnp.where(tot > 0, lt - ft + 1, 0)
        te = jnp.cumsum(ntl, axis=0)
        ts = te - ntl
        total = _lane(te, E - 1)
        ft_v[...] = ft
        ts_v[...] = ts
        st_v[...] = starts
        en_v[...] = ends
        last = total - 1
        g_last = jnp.sum(jnp.where(te <= last, 1, 0), axis=0)
        m_last = _lane(ft, g_last) + (last - _lane(ts, g_last))
        for half in range(2):
            sv = half * L + lanes
            gv = jnp.zeros((L,), jnp.int32)
            for gp in range(E):
                gv = gv + jnp.where(_lane(te, gp) <= sv, 1, 0)
            gv = jnp.minimum(gv, E - 1)
            mv = plsc.load_gather(ft_v, [gv]) + sv - plsc.load_gather(ts_v, [gv])
            lov = jnp.maximum(plsc.load_gather(st_v, [gv]), mv * TM)
            hiv = jnp.minimum(plsc.load_gather(en_v, [gv]), (mv + 1) * TM)
            valid = sv < total
            mv = jnp.where(valid, mv, m_last)
            gv = jnp.where(valid, gv, g_last)
            lov = jnp.where(valid, lov, 0)
            hiv = jnp.where(valid, hiv, 0)
            meta_v[0, pl.ds(half * L, L)] = mv
            meta_v[1, pl.ds(half * L, L)] = gv
            meta_v[2, pl.ds(half * L, L)] = lov
            meta_v[3, pl.ds(half * L, L)] = hiv
        pltpu.sync_copy(meta_v, meta_out)


def _dispatch(e_ids, x_flat):
    mesh = plsc.VectorSubcoreMesh(
        core_axis_name="c", subcore_axis_name="s", num_cores=1)
    fn = functools.partial(
        pl.kernel,
        out_type=[
            jax.ShapeDtypeStruct((N_TOK, D), jnp.float32),
            jax.ShapeDtypeStruct((N_TOK,), jnp.int32),
            jax.ShapeDtypeStruct((4, 2 * L), jnp.int32),
        ],
        scratch_types=[
            pltpu.VMEM((SORT_CHUNK,), jnp.int32),
            pltpu.VMEM((SORT_NW, L), jnp.int32),
            pltpu.VMEM((L,), jnp.int32),
            pltpu.VMEM((SORT_CHUNK,), jnp.int32),
            pltpu.VMEM((SORT_CHUNK,), jnp.int32),
            pltpu.VMEM((SORT_CHUNK,), jnp.int32),
            pltpu.VMEM((SORT_CHUNK, D), jnp.float32),
            pltpu.VMEM((L,), jnp.int32),
            pltpu.VMEM((L,), jnp.int32),
            pltpu.VMEM((L,), jnp.int32),
            pltpu.VMEM((L,), jnp.int32),
            pltpu.VMEM((4, 2 * L), jnp.int32),
            pltpu.VMEM_SHARED((SORT_NW, L), jnp.int32),
            pltpu.VMEM_SHARED((N_TOK,), jnp.int32),
            pltpu.SemaphoreType.DMA,
            pltpu.SemaphoreType.DMA,
        ],
        mesh=mesh,
        compiler_params=pltpu.CompilerParams(needs_layout_passes=False),
    )(_dispatch_body)
    return fn(e_ids, x_flat)


def _combine_body(y_hbm, inv_hbm, out_hbm, iv_v, rows_v, sem, wsem):
    wid = lax.axis_index("s") * 2 + lax.axis_index("c")
    base = wid * COMB_CHUNK
    pltpu.sync_copy(inv_hbm.at[pl.ds(base, COMB_CHUNK)], iv_v)
    nch = COMB_CHUNK // L
    gcopies = []
    for k in range(nch):
        ivk = iv_v[pl.ds(k * L, L)]
        gcopies.append(pltpu.async_copy(
            y_hbm.at[ivk], rows_v.at[pl.ds(k * L, L)], sem))
    wcopies = []
    for k in range(nch):
        gcopies[k].wait()
        wcopies.append(pltpu.async_copy(
            rows_v.at[pl.ds(k * L, L)],
            out_hbm.at[pl.ds(base + k * L, L)], wsem))
    for c in wcopies:
        c.wait()


def _combine(y_sorted, inv):
    mesh = plsc.VectorSubcoreMesh(core_axis_name="c", subcore_axis_name="s")
    fn = functools.partial(
        pl.kernel,
        out_type=jax.ShapeDtypeStruct((N_TOK, D), jnp.float32),
        scratch_types=[
            pltpu.VMEM((COMB_CHUNK,), jnp.int32),
            pltpu.VMEM((COMB_CHUNK, D), jnp.float32),
            pltpu.SemaphoreType.DMA,
            pltpu.SemaphoreType.DMA,
        ],
        mesh=mesh,
        compiler_params=pltpu.CompilerParams(needs_layout_passes=False),
    )(_combine_body)
    return fn(y_sorted, inv)


def _router_body(x_ref, wr_ref, out_ref):
    logits = lax.dot_general(
        x_ref[...], wr_ref[...], (((1,), (1,)), ((), ())),
        preferred_element_type=jnp.float32)
    am = jnp.argmax(logits, axis=1).astype(jnp.int32)
    out_ref[...] = am.reshape(TM, 1)


def _router_argmax(x_flat, Wr):
    return pl.pallas_call(
        _router_body,
        grid=(NMT,),
        in_specs=[
            pl.BlockSpec((TM, D), lambda i: (i, 0)),
            pl.BlockSpec((E, D), lambda i: (0, 0)),
        ],
        out_specs=pl.BlockSpec((TM, 1), lambda i: (i, 0)),
        out_shape=jax.ShapeDtypeStruct((N_TOK, 1), jnp.int32),
    )(x_flat, Wr)


def _ffn_body(meta_ref, x_ref, w1_ref, b1_ref, w2_ref, b2_ref, out_ref):
    t = pl.program_id(0)
    m = meta_ref[0, t]
    lo = meta_ref[2, t]
    hi = meta_ref[3, t]
    prev_m = meta_ref[0, jnp.maximum(t - 1, 0)]
    first = (t == 0) | (m != prev_m)

    @pl.when(first)
    def _():
        out_ref[...] = jnp.zeros_like(out_ref)

    @pl.when(lo < hi)
    def _():
        xb = x_ref[...]
        w1 = w1_ref[0]
        hb = lax.dot_general(xb, w1, (((1,), (1,)), ((), ())),
                             preferred_element_type=jnp.float32)
        hb = jnp.maximum(hb + b1_ref[0, 0][None, :], 0.0)
        rows = m * TM + lax.broadcasted_iota(jnp.int32, (TM, 1), 0)
        mask = (rows >= lo) & (rows < hi)
        hb = jnp.where(mask, hb, 0.0)
        w2 = w2_ref[0]
        yb = lax.dot_general(hb, w2, (((1,), (1,)), ((), ())),
                             preferred_element_type=jnp.float32)
        yb = yb + jnp.where(mask, b2_ref[0, 0][None, :], 0.0)
        out_ref[...] += yb


def _grouped_ffn(x_sorted, W1, b1, W2, b2, meta):
    grid_spec = pltpu.PrefetchScalarGridSpec(
        num_scalar_prefetch=1,
        grid=(T_MAX,),
        in_specs=[
            pl.BlockSpec((TM, D), lambda t, md: (md[0, t], 0)),
            pl.BlockSpec((1, H, D), lambda t, md: (md[1, t], 0, 0)),
            pl.BlockSpec((1, 1, H), lambda t, md: (md[1, t], 0, 0)),
            pl.BlockSpec((1, D, H), lambda t, md: (md[1, t], 0, 0)),
            pl.BlockSpec((1, 1, D), lambda t, md: (md[1, t], 0, 0)),
        ],
        out_specs=pl.BlockSpec((TM, D), lambda t, md: (md[0, t], 0)),
    )
    return pl.pallas_call(
        _ffn_body,
        grid_spec=grid_spec,
        out_shape=jax.ShapeDtypeStruct((N_TOK, D), jnp.float32),
    )(meta, x_sorted, W1, b1.reshape(E, 1, H), W2, b2.reshape(E, 1, D))


def kernel(x, Wr, W1, b1, W2, b2):
    Bc, Tc, C = x.shape
    x_flat = x.reshape(-1, C)
    e_ids = _router_argmax(x_flat, Wr).reshape(N_TOK)
    x_sorted, inv, meta = _dispatch(e_ids, x_flat)
    y_sorted = _grouped_ffn(x_sorted, W1, b1, W2, b2, meta)
    out = _combine(y_sorted, inv)
    return out.reshape(Bc, Tc, C)
